# baseline (speedup 1.0000x reference)
import jax
import jax.numpy as jnp
from jax import lax
from jax.experimental import pallas as pl
from jax.experimental.pallas import tpu as pltpu

N = 1024
D = 128
L = 64
TI = 128
TJ = 128


def _prep_body(x_ref, w1_ref, b1_ref, w2c_ref, w2r_ref, wc_ref, b2_ref,
               hi_ref, hjt_ref, li_ref, lj_ref, wcs_ref, s_ref, cw_ref):
    x = x_ref[...]
    wa = w1_ref[0:D, :]
    wb = w1_ref[D:2 * D, :]
    b1 = b1_ref[...]
    w2c = w2c_ref[...]
    w2r = w2r_ref[...]
    wc = wc_ref[...]

    hi = jnp.dot(x, wa, preferred_element_type=jnp.float32) + b1
    hjt = lax.dot_general(wb, x, (((0,), (1,)), ((), ())),
                          preferred_element_type=jnp.float32)

    scale_r = 0.495 * jnp.abs(w2r)
    scale_c = 0.495 * jnp.abs(w2c)
    hi_ref[...] = hi * scale_r
    hjt_ref[...] = hjt * scale_c
    li_ref[...] = 0.505 * jnp.dot(hi, w2c, preferred_element_type=jnp.float32) + b2_ref[0, 0]
    lj_ref[...] = 0.505 * lax.dot_general(w2c, hjt, (((0,), (0,)), ((), ())),
                                          preferred_element_type=jnp.float32)
    wcs_ref[...] = scale_r * wc
    s_ref[...] = jnp.sign(w2r)
    cw_ref[...] = 0.505 * jnp.sum(w2r * wc, axis=(0, 1), keepdims=True)


def _edge_body(hi_ref, hjt_ref, adj_ref, li_ref, lj_ref, wcs_ref, s_ref,
               cw_ref, out_ref):
    adj = adj_ref[...]
    acc = li_ref[...] + lj_ref[...] + cw_ref[...] * adj
    for l in range(L):
        h = hi_ref[:, l:l + 1] + hjt_ref[l:l + 1, :] + wcs_ref[0:1, l:l + 1] * adj
        acc = acc + s_ref[0:1, l:l + 1] * jnp.abs(h)
    out_ref[...] = jnp.maximum(acc, 0.0)


def kernel(x, in_adj, temp, W1, b1, W2, b2):
    del temp
    b1r = b1.reshape(1, L)
    w2c = W2.reshape(L, 1)
    w2r = W2.reshape(1, L)
    wcr = W1[2 * D].reshape(1, L)
    b2m = b2.reshape(1, 1)

    hi, hjt, li, lj, wcs, s, cw = pl.pallas_call(
        _prep_body,
        out_shape=[
            jax.ShapeDtypeStruct((N, L), jnp.float32),
            jax.ShapeDtypeStruct((L, N), jnp.float32),
            jax.ShapeDtypeStruct((N, 1), jnp.float32),
            jax.ShapeDtypeStruct((1, N), jnp.float32),
            jax.ShapeDtypeStruct((1, L), jnp.float32),
            jax.ShapeDtypeStruct((1, L), jnp.float32),
            jax.ShapeDtypeStruct((1, 1), jnp.float32),
        ],
    )(x, W1, b1r, w2c, w2r, wcr, b2m)

    out = pl.pallas_call(
        _edge_body,
        grid=(N // TI, N // TJ),
        in_specs=[
            pl.BlockSpec((TI, L), lambda i, j: (i, 0)),
            pl.BlockSpec((L, TJ), lambda i, j: (0, j)),
            pl.BlockSpec((TI, TJ), lambda i, j: (i, j)),
            pl.BlockSpec((TI, 1), lambda i, j: (i, 0)),
            pl.BlockSpec((1, TJ), lambda i, j: (0, j)),
            pl.BlockSpec((1, L), lambda i, j: (0, 0)),
            pl.BlockSpec((1, L), lambda i, j: (0, 0)),
            pl.BlockSpec((1, 1), lambda i, j: (0, 0)),
        ],
        out_specs=pl.BlockSpec((TI, TJ), lambda i, j: (i, j)),
        out_shape=jax.ShapeDtypeStruct((N, N), jnp.float32),
        compiler_params=pltpu.CompilerParams(
            dimension_semantics=("parallel", "parallel"),
        ),
    )(hi, hjt, in_adj, li, lj, wcs, s, cw)

    return out[None, :, :]


# speedup vs baseline: 1.7942x; 1.7942x over previous
import jax
import jax.numpy as jnp
from jax import lax
from jax.experimental import pallas as pl
from jax.experimental.pallas import tpu as pltpu

N = 1024
D = 128
L = 64
TI = 16
TJ = 1024


def _body(x_ref, w1_ref, b1_ref, w2c_ref, w2r_ref, wcr_ref, wcc_ref, b2_ref,
          adj_ref, out_ref, u_ref, vt_ref, li_ref, lj_ref, t_ref, cw_ref):
    i = pl.program_id(0)

    @pl.when(i == 0)
    def _prep():
        x = x_ref[...]
        wa = w1_ref[0:D, :]
        wb = w1_ref[D:2 * D, :]
        b1 = b1_ref[...]
        w2c = w2c_ref[...]
        w2r = w2r_ref[...]
        wcr = wcr_ref[...]
        wcc = wcc_ref[...]

        hi = jnp.dot(x, wa, preferred_element_type=jnp.float32) + b1
        hjt = lax.dot_general(wb, x, (((0,), (1,)), ((), ())),
                              preferred_element_type=jnp.float32)

        wcr_s = jnp.where(jnp.abs(wcr) < 1e-20, 1e-20, wcr)
        wcc_s = jnp.where(jnp.abs(wcc) < 1e-20, 1e-20, wcc)
        u_ref[...] = hi * (1.0 / wcr_s)
        vt = hjt * (1.0 / wcc_s)
        vt_ref[...] = jnp.broadcast_to(vt[:, None, :], (L, TI, N)).reshape(L * TI, N)
        li_ref[...] = 0.505 * jnp.dot(hi, w2c, preferred_element_type=jnp.float32) + b2_ref[0, 0]
        lj_ref[...] = 0.505 * lax.dot_general(w2c, hjt, (((0,), (0,)), ((), ())),
                                              preferred_element_type=jnp.float32)
        t_ref[...] = 0.495 * jnp.sign(w2r) * jnp.abs(w2r * wcr_s)
        cw_ref[...] = 0.505 * jnp.sum(w2r * wcr, axis=(0, 1), keepdims=True)

    r0 = pl.multiple_of(i * TI, TI)
    adj = adj_ref[...]
    acc = li_ref[pl.ds(r0, TI), :] + lj_ref[...] + cw_ref[...] * adj
    for l in range(L):
        m = adj + (u_ref[pl.ds(r0, TI), l:l + 1] + vt_ref[l * TI:(l + 1) * TI, :])
        acc = acc + t_ref[0:1, l:l + 1] * jnp.abs(m)
    out_ref[...] = jnp.maximum(acc, 0.0)


def kernel(x, in_adj, temp, W1, b1, W2, b2):
    del temp
    b1r = b1.reshape(1, L)
    w2c = W2.reshape(L, 1)
    w2r = W2.reshape(1, L)
    wcr = W1[2 * D].reshape(1, L)
    wcc = W1[2 * D].reshape(L, 1)
    b2m = b2.reshape(1, 1)

    full = lambda s: pl.BlockSpec(s, lambda i: tuple(0 for _ in s))
    out = pl.pallas_call(
        _body,
        grid=(N // TI,),
        in_specs=[
            full((N, D)),
            full((2 * D + 1, L)),
            full((1, L)),
            full((L, 1)),
            full((1, L)),
            full((1, L)),
            full((L, 1)),
            full((1, 1)),
            pl.BlockSpec((TI, TJ), lambda i: (i, 0)),
        ],
        out_specs=pl.BlockSpec((TI, TJ), lambda i: (i, 0)),
        out_shape=jax.ShapeDtypeStruct((N, N), jnp.float32),
        scratch_shapes=[
            pltpu.VMEM((N, L), jnp.float32),
            pltpu.VMEM((L * TI, N), jnp.float32),
            pltpu.VMEM((N, 1), jnp.float32),
            pltpu.VMEM((1, N), jnp.float32),
            pltpu.VMEM((1, L), jnp.float32),
            pltpu.VMEM((1, 1), jnp.float32),
        ],
        compiler_params=pltpu.CompilerParams(
            dimension_semantics=("arbitrary",),
        ),
    )(x, W1, b1r, w2c, w2r, wcr, wcc, b2m, in_adj)

    return out[None, :, :]
